# baseline (speedup 1.0000x reference)
import functools

import jax
import jax.numpy as jnp
from jax import lax
from jax.experimental import pallas as pl
from jax.experimental.pallas import tpu as pltpu
from jax.experimental.pallas import tpu_sc as plsc

B, S, E, V = 4, 2048, 768, 100000
NC, NS, L = 2, 16, 16
NW = NC * NS
SCHUNK = S // NW
EV = E // L
CS = 8
NCHUNK = SCHUNK // CS
CH = B * CS
NBUF = 3


def _make_kernel():
    mesh = plsc.VectorSubcoreMesh(core_axis_name="c", subcore_axis_name="s")

    @functools.partial(
        pl.kernel,
        out_type=jax.ShapeDtypeStruct((B, S, E), jnp.float32),
        mesh=mesh,
        scratch_types=[
            pltpu.VMEM((NCHUNK, CH), jnp.int32),
            pltpu.VMEM((SCHUNK, E), jnp.float32),
            [pltpu.VMEM((CH, E), jnp.float32) for _ in range(NBUF)],
            [pltpu.SemaphoreType.DMA for _ in range(NBUF)],
            [pltpu.SemaphoreType.DMA for _ in range(NBUF)],
            pltpu.SemaphoreType.DMA,
            [pltpu.SemaphoreType.DMA for _ in range(NCHUNK)],
        ],
    )
    def k(ids_hbm, tab_hbm, pos_hbm, out_hbm, idx_v, pos_v, bufs, gsems, wsems,
          psem, isems):
        wid = lax.axis_index("s") * NC + lax.axis_index("c")
        s0 = wid * SCHUNK

        pos_cp = pltpu.async_copy(pos_hbm.at[pl.ds(s0, SCHUNK)], pos_v, psem)
        i_cp = [
            [
                pltpu.async_copy(
                    ids_hbm.at[b, pl.ds(s0 + c * CS, CS)],
                    idx_v.at[c, pl.ds(b * CS, CS)],
                    isems[c],
                )
                for b in range(B)
            ]
            for c in range(NCHUNK)
        ]

        def gather(c):
            for cp in i_cp[c]:
                cp.wait()
            return pltpu.async_copy(
                tab_hbm.at[idx_v.at[c]],
                bufs[c % NBUF],
                gsems[c % NBUF],
            )

        def write(c):
            for b in range(B):
                pltpu.async_copy(
                    bufs[c % NBUF].at[pl.ds(b * CS, CS)],
                    out_hbm.at[b, pl.ds(s0 + c * CS, CS)],
                    wsems[c % NBUF],
                )
            return pltpu.make_async_copy(
                out_hbm.at[0, pl.ds(s0, CH)],
                bufs[c % NBUF],
                wsems[c % NBUF],
            )

        g_cp = [None] * NCHUNK
        w_cp = [None] * NCHUNK
        g_cp[0] = gather(0)
        g_cp[1] = gather(1)
        pos_cp.wait()

        for c in range(NCHUNK):
            g_cp[c].wait()

            buf = bufs[c % NBUF]

            @plsc.parallel_loop(0, CS, 1)
            def add_row(sl):
                pr = c * CS + sl
                for e in range(EV):
                    pv = pos_v[pr, pl.ds(e * L, L)]
                    for b in range(B):
                        plsc.addupdate(
                            buf.at[b * CS + sl, pl.ds(e * L, L)], pv
                        )

            nc = c + 2
            if nc < NCHUNK:
                if c >= 1:
                    w_cp[c - 1].wait()
                g_cp[nc] = gather(nc)

            w_cp[c] = write(c)

        for c in (NCHUNK - 3, NCHUNK - 2, NCHUNK - 1):
            w_cp[c].wait()

    return k


_kernel = _make_kernel()


def kernel(input_ids, token_embeddings, position_embeddings):
    return _kernel(input_ids.astype(jnp.int32), token_embeddings,
                   position_embeddings)


# speedup vs baseline: 1.0028x; 1.0028x over previous
import functools

import jax
import jax.numpy as jnp
from jax import lax
from jax.experimental import pallas as pl
from jax.experimental.pallas import tpu as pltpu
from jax.experimental.pallas import tpu_sc as plsc

B, S, E, V = 4, 2048, 768, 100000
NC, NS, L = 2, 16, 16
NW = NC * NS
SCHUNK = S // NW
EV = E // L
CS = 8
NCHUNK = SCHUNK // CS
CH = B * CS
NBUF = 3


def _make_kernel():
    mesh = plsc.VectorSubcoreMesh(core_axis_name="c", subcore_axis_name="s")

    @functools.partial(
        pl.kernel,
        out_type=jax.ShapeDtypeStruct((B, S, E), jnp.float32),
        mesh=mesh,
        scratch_types=[
            pltpu.VMEM((NCHUNK, CH), jnp.int32),
            pltpu.VMEM((SCHUNK, E), jnp.float32),
            [pltpu.VMEM((CH, E), jnp.float32) for _ in range(NBUF)],
            [pltpu.SemaphoreType.DMA for _ in range(NBUF)],
            [pltpu.SemaphoreType.DMA for _ in range(NBUF)],
            pltpu.SemaphoreType.DMA,
            [pltpu.SemaphoreType.DMA for _ in range(NCHUNK)],
        ],
    )
    def k(ids_hbm, tab_hbm, pos_hbm, out_hbm, idx_v, pos_v, bufs, gsems, wsems,
          psem, isems):
        wid = lax.axis_index("s") * NC + lax.axis_index("c")
        s0 = wid * SCHUNK

        pos_cp = pltpu.async_copy(pos_hbm.at[pl.ds(s0, SCHUNK)], pos_v, psem)
        i_cp = [
            [
                pltpu.async_copy(
                    ids_hbm.at[b, pl.ds(s0 + c * CS, CS)],
                    idx_v.at[c, pl.ds(b * CS, CS)],
                    isems[c],
                )
                for b in range(B)
            ]
            for c in range(NCHUNK)
        ]

        def gather(c):
            for cp in i_cp[c]:
                cp.wait()
            return pltpu.async_copy(
                tab_hbm.at[idx_v.at[c]],
                bufs[c % NBUF],
                gsems[c % NBUF],
            )

        def write(c):
            for b in range(B):
                pltpu.async_copy(
                    bufs[c % NBUF].at[pl.ds(b * CS, CS)],
                    out_hbm.at[b, pl.ds(s0 + c * CS, CS)],
                    wsems[c % NBUF],
                )
            return pltpu.make_async_copy(
                out_hbm.at[0, pl.ds(s0, CH)],
                bufs[c % NBUF],
                wsems[c % NBUF],
            )

        g_cp = [None] * NCHUNK
        w_cp = [None] * NCHUNK
        g_cp[0] = gather(0)
        g_cp[1] = gather(1)
        pos_cp.wait()

        for c in range(NCHUNK):
            g_cp[c].wait()

            buf = bufs[c % NBUF]

            @plsc.parallel_loop(0, CS, 1)
            def add_row(sl):
                pr = c * CS + sl
                for e in range(EV):
                    pv = pos_v[pr, pl.ds(e * L, L)]
                    for b in range(B):
                        plsc.addupdate(
                            buf.at[b * CS + sl, pl.ds(e * L, L)], pv
                        )

            w_cp[c] = write(c)

            nc = c + 2
            if nc < NCHUNK:
                if c >= 1:
                    w_cp[c - 1].wait()
                g_cp[nc] = gather(nc)

        for c in (NCHUNK - 3, NCHUNK - 2, NCHUNK - 1):
            w_cp[c].wait()

    return k


_kernel = _make_kernel()


def kernel(input_ids, token_embeddings, position_embeddings):
    return _kernel(input_ids.astype(jnp.int32), token_embeddings,
                   position_embeddings)


# speedup vs baseline: 1.0148x; 1.0120x over previous
import functools

import jax
import jax.numpy as jnp
from jax import lax
from jax.experimental import pallas as pl
from jax.experimental.pallas import tpu as pltpu
from jax.experimental.pallas import tpu_sc as plsc

B, S, E, V = 4, 2048, 768, 100000
NC, NS, L = 2, 16, 16
NW = NC * NS
SCHUNK = S // NW
EV = E // L
CS = 8
BB = 2
NQ = SCHUNK // CS
NCHUNK = NQ * (B // BB)
CH = BB * CS
NBUF = 6
PRIME = 4


def _make_kernel():
    mesh = plsc.VectorSubcoreMesh(core_axis_name="c", subcore_axis_name="s")

    @functools.partial(
        pl.kernel,
        out_type=jax.ShapeDtypeStruct((B, S, E), jnp.float32),
        mesh=mesh,
        scratch_types=[
            pltpu.VMEM((NQ, B * CS), jnp.int32),
            pltpu.VMEM((SCHUNK, E), jnp.float32),
            [pltpu.VMEM((CH, E), jnp.float32) for _ in range(NBUF)],
            [pltpu.SemaphoreType.DMA for _ in range(NBUF)],
            [pltpu.SemaphoreType.DMA for _ in range(NBUF)],
            pltpu.SemaphoreType.DMA,
            [pltpu.SemaphoreType.DMA for _ in range(NQ)],
        ],
    )
    def k(ids_hbm, tab_hbm, pos_hbm, out_hbm, idx_v, pos_v, bufs, gsems, wsems,
          psem, isems):
        wid = lax.axis_index("s") * NC + lax.axis_index("c")
        s0 = wid * SCHUNK

        pos_cp = pltpu.async_copy(pos_hbm.at[pl.ds(s0, SCHUNK)], pos_v, psem)
        i_cp = [
            [
                pltpu.async_copy(
                    ids_hbm.at[b, pl.ds(s0 + q * CS, CS)],
                    idx_v.at[q, pl.ds(b * CS, CS)],
                    isems[q],
                )
                for b in range(B)
            ]
            for q in range(NQ)
        ]

        def gather(c):
            q, half = c // 2, c % 2
            if half == 0:
                for cp in i_cp[q]:
                    cp.wait()
            return pltpu.async_copy(
                tab_hbm.at[idx_v.at[q, pl.ds(half * CH, CH)]],
                bufs[c % NBUF],
                gsems[c % NBUF],
            )

        def write(c):
            q, half = c // 2, c % 2
            for j in range(BB):
                pltpu.async_copy(
                    bufs[c % NBUF].at[pl.ds(j * CS, CS)],
                    out_hbm.at[half * BB + j, pl.ds(s0 + q * CS, CS)],
                    wsems[c % NBUF],
                )
            return pltpu.make_async_copy(
                out_hbm.at[0, pl.ds(s0, CH)],
                bufs[c % NBUF],
                wsems[c % NBUF],
            )

        g_cp = [None] * NCHUNK
        w_cp = [None] * NCHUNK
        for c in range(PRIME):
            g_cp[c] = gather(c)
        pos_cp.wait()

        for c in range(NCHUNK):
            g_cp[c].wait()

            q = c // 2
            buf = bufs[c % NBUF]

            @plsc.parallel_loop(0, CS, 1)
            def add_row(sl):
                pr = q * CS + sl
                for e in range(EV):
                    pv = pos_v[pr, pl.ds(e * L, L)]
                    for j in range(BB):
                        plsc.addupdate(
                            buf.at[j * CS + sl, pl.ds(e * L, L)], pv
                        )

            w_cp[c] = write(c)

            nc = c + PRIME
            if nc < NCHUNK:
                wb = nc - NBUF
                if wb >= 0:
                    w_cp[wb].wait()
                g_cp[nc] = gather(nc)

        for c in range(NCHUNK - NBUF, NCHUNK):
            w_cp[c].wait()

    return k


_kernel = _make_kernel()


def kernel(input_ids, token_embeddings, position_embeddings):
    return _kernel(input_ids.astype(jnp.int32), token_embeddings,
                   position_embeddings)
